# Initial kernel scaffold; baseline (speedup 1.0000x reference)
import jax
import jax.numpy as jnp
from jax.experimental import pallas as pl


def kernel(x_user, x_item, W_l_u2i, W_r_u2i, b_u2i, W_l_i2u, W_r_i2u, b_i2u, edge_index_u2i, edge_index_i2u):
    raise NotImplementedError("write your pallas kernel here")


# speedup vs baseline: 3.0181x; 3.0181x over previous
import functools

import jax
import jax.numpy as jnp
from jax import lax
from jax.experimental import pallas as pl
from jax.experimental.pallas import tpu as pltpu
from jax.experimental.pallas import tpu_sc as plsc

_N = 10000
_D = 256
_DH = _D // 2
_E = 160000
_NS = 16
_K = 80
_SEG = 25
_NSEG = 5
_RPT = _N // _NS


def _sc_agg_body(xh, srcr, dstr, agg_out, cnt_out, idx_v, dstb, gbuf,
                 agg_sp, gsem):
    c = lax.axis_index("c")
    s = lax.axis_index("s")
    zero16 = jnp.zeros((16,), jnp.float32)
    one16 = jnp.ones((16,), jnp.float32)

    def fill(val):
        def fz(i, _):
            for jj in range(_DH // 16):
                gbuf[i, pl.ds(jj * 16, 16)] = val
            return 0
        lax.fori_loop(0, _K, fz, 0)

    def zero_my_rows():
        def fzs(i, _):
            pltpu.sync_copy(gbuf, agg_sp.at[pl.ds(s * _RPT + i * _K, _K)])
            return 0
        lax.fori_loop(0, _RPT // _K, fzs, 0)
        rem = _RPT - (_RPT // _K) * _K
        pltpu.sync_copy(gbuf.at[pl.ds(0, rem)],
                        agg_sp.at[pl.ds(s * _RPT + (_RPT // _K) * _K, rem)])

    fill(zero16)
    zero_my_rows()
    plsc.subcore_barrier()

    def fseg(g, _):
        pltpu.sync_copy(srcr.at[s, g], idx_v)

        def ft(i, _):
            for jj in range(_K // 16):
                v = idx_v[i, pl.ds(jj * 16, 16)]
                idx_v[i, pl.ds(jj * 16, 16)] = v * 2 + c
            return 0
        lax.fori_loop(0, _SEG, ft, 0)

        def fb(b, _):
            pltpu.async_copy(xh.at[idx_v.at[b]], gbuf, gsem).wait()
            pltpu.sync_copy(dstr.at[s, g, b], dstb)
            pltpu.sync_copy(gbuf, agg_sp.at[dstb], add=True)
            return 0
        lax.fori_loop(0, _SEG, fb, 0)
        return 0
    lax.fori_loop(0, _NSEG, fseg, 0)

    plsc.subcore_barrier()
    pltpu.sync_copy(agg_sp.at[pl.ds(s * _RPT, _RPT)], agg_out.at[c * _NS + s])
    plsc.subcore_barrier()

    fill(zero16)
    zero_my_rows()
    fill(one16)
    plsc.subcore_barrier()

    def cseg(g, _):
        def cb(b, _):
            @pl.when(b % 2 == c)
            def _():
                pltpu.sync_copy(dstr.at[s, g, b], dstb)
                pltpu.sync_copy(gbuf, agg_sp.at[dstb], add=True)
            return 0
        lax.fori_loop(0, _SEG, cb, 0)
        return 0
    lax.fori_loop(0, _NSEG, cseg, 0)

    plsc.subcore_barrier()
    pltpu.sync_copy(agg_sp.at[pl.ds(s * _RPT, _RPT)], cnt_out.at[c * _NS + s])


_sc_aggregate = functools.partial(
    pl.kernel,
    out_type=(jax.ShapeDtypeStruct((2 * _NS, _RPT, _DH), jnp.float32),
              jax.ShapeDtypeStruct((2 * _NS, _RPT, _DH), jnp.float32)),
    mesh=plsc.VectorSubcoreMesh(core_axis_name="c", subcore_axis_name="s"),
    scratch_types=[
        pltpu.VMEM((_SEG, _K), jnp.int32),
        pltpu.VMEM((_K,), jnp.int32),
        pltpu.VMEM((_K, _DH), jnp.float32),
        pltpu.VMEM_SHARED((_N, _DH), jnp.float32),
        pltpu.SemaphoreType.DMA,
    ],
)(_sc_agg_body)


_BM = 1000


def _tc_body(a0, a1, c0, c1, xd, wl0, wl1, wr, bias, out):
    r = 1.0 / jnp.maximum(c0[:, 0:1] + c1[:, 0:1], 1.0)
    m = jnp.dot(a0[...] * r, wl0[...], preferred_element_type=jnp.float32)
    m += jnp.dot(a1[...] * r, wl1[...], preferred_element_type=jnp.float32)
    m += jnp.dot(xd[...], wr[...], preferred_element_type=jnp.float32)
    out[...] = m + bias[...]


def _fused_update(aggflat, cntflat, x_dst, W_l, W_r, b):
    nblk = _N // _BM
    return pl.pallas_call(
        _tc_body,
        grid=(nblk,),
        in_specs=[
            pl.BlockSpec((_BM, _DH), lambda i: (i, 0)),
            pl.BlockSpec((_BM, _DH), lambda i: (i + _N // _BM, 0)),
            pl.BlockSpec((_BM, _DH), lambda i: (i, 0)),
            pl.BlockSpec((_BM, _DH), lambda i: (i + _N // _BM, 0)),
            pl.BlockSpec((_BM, _D), lambda i: (i, 0)),
            pl.BlockSpec((_DH, _D), lambda i: (0, 0)),
            pl.BlockSpec((_DH, _D), lambda i: (0, 0)),
            pl.BlockSpec((_D, _D), lambda i: (0, 0)),
            pl.BlockSpec((1, _D), lambda i: (0, 0)),
        ],
        out_specs=pl.BlockSpec((_BM, _D), lambda i: (i, 0)),
        out_shape=jax.ShapeDtypeStruct((_N, _D), jnp.float32),
    )(aggflat, aggflat, cntflat, cntflat, x_dst,
      W_l[:_DH], W_l[_DH:], W_r, b.reshape(1, _D))


def _sage(x_src, x_dst, edge_index, W_l, W_r, b):
    xh = x_src.reshape(2 * _N, _DH)
    srcr = edge_index[0].reshape(_NS, _NSEG, _SEG, _K)
    dstr = edge_index[1].reshape(_NS, _NSEG, _SEG, _K)
    agg3, cnt3 = _sc_aggregate(xh, srcr, dstr)
    aggflat = agg3.reshape(2 * _N, _DH)
    cntflat = cnt3.reshape(2 * _N, _DH)
    return _fused_update(aggflat, cntflat, x_dst, W_l, W_r, b)


def kernel(x_user, x_item, W_l_u2i, W_r_u2i, b_u2i, W_l_i2u, W_r_i2u, b_i2u,
           edge_index_u2i, edge_index_i2u):
    out_item = _sage(x_user, x_item, edge_index_u2i, W_l_u2i, W_r_u2i, b_u2i)
    out_user = _sage(x_item, x_user, edge_index_i2u, W_l_i2u, W_r_i2u, b_i2u)
    return (out_user, out_item)


# speedup vs baseline: 4.4645x; 1.4792x over previous
import functools

import jax
import jax.numpy as jnp
from jax import lax
from jax.experimental import pallas as pl
from jax.experimental.pallas import tpu as pltpu
from jax.experimental.pallas import tpu_sc as plsc

_N = 10000
_D = 256
_DH = _D // 2
_E = 160000
_NS = 16
_K = 80
_SEG = 25
_NSEG = 5
_RPT = _N // _NS


def _sc_agg_body(xh, srcr, dstr, agg_out, cnt_out, idx_v, dstb, gb0, gb1,
                 agg_sp, sem0, sem1):
    c = lax.axis_index("c")
    s = lax.axis_index("s")
    zero16 = jnp.zeros((16,), jnp.float32)
    one16 = jnp.ones((16,), jnp.float32)

    def fill(val):
        def fz(i, _):
            for jj in range(_DH // 16):
                gb0[i, pl.ds(jj * 16, 16)] = val
            return 0
        lax.fori_loop(0, _K, fz, 0)

    def zero_my_rows():
        def fzs(i, _):
            pltpu.sync_copy(gb0, agg_sp.at[pl.ds(s * _RPT + i * _K, _K)])
            return 0
        lax.fori_loop(0, _RPT // _K, fzs, 0)
        rem = _RPT - (_RPT // _K) * _K
        pltpu.sync_copy(gb0.at[pl.ds(0, rem)],
                        agg_sp.at[pl.ds(s * _RPT + (_RPT // _K) * _K, rem)])

    fill(zero16)
    zero_my_rows()
    plsc.subcore_barrier()

    def fseg(g, _):
        pltpu.sync_copy(srcr.at[s, g], idx_v)

        def ft(i, _):
            for jj in range(_K // 16):
                v = idx_v[i, pl.ds(jj * 16, 16)]
                idx_v[i, pl.ds(jj * 16, 16)] = v * 2 + c
            return 0
        lax.fori_loop(0, _SEG, ft, 0)

        def fb(b, _):
            even = (b % 2) == 0

            @pl.when(b == 0)
            def _():
                pltpu.async_copy(xh.at[idx_v.at[b]], gb0, sem0)

            @pl.when(jnp.logical_and(b + 1 < _SEG, even))
            def _():
                pltpu.async_copy(xh.at[idx_v.at[b + 1]], gb1, sem1)

            @pl.when(jnp.logical_and(b + 1 < _SEG, jnp.logical_not(even)))
            def _():
                pltpu.async_copy(xh.at[idx_v.at[b + 1]], gb0, sem0)

            pltpu.sync_copy(dstr.at[s, g, b], dstb)

            @pl.when(even)
            def _():
                pltpu.make_async_copy(xh.at[pl.ds(0, _K)], gb0, sem0).wait()
                pltpu.sync_copy(gb0, agg_sp.at[dstb], add=True)

            @pl.when(jnp.logical_not(even))
            def _():
                pltpu.make_async_copy(xh.at[pl.ds(0, _K)], gb1, sem1).wait()
                pltpu.sync_copy(gb1, agg_sp.at[dstb], add=True)
            return 0
        lax.fori_loop(0, _SEG, fb, 0)
        return 0
    lax.fori_loop(0, _NSEG, fseg, 0)

    plsc.subcore_barrier()
    pltpu.sync_copy(agg_sp.at[pl.ds(s * _RPT, _RPT)], agg_out.at[c * _NS + s])
    plsc.subcore_barrier()

    fill(zero16)
    zero_my_rows()
    fill(one16)
    plsc.subcore_barrier()

    def cseg(g, _):
        def cb(b, _):
            @pl.when(b % 2 == c)
            def _():
                pltpu.sync_copy(dstr.at[s, g, b], dstb)
                pltpu.sync_copy(gb0, agg_sp.at[dstb], add=True)
            return 0
        lax.fori_loop(0, _SEG, cb, 0)
        return 0
    lax.fori_loop(0, _NSEG, cseg, 0)

    plsc.subcore_barrier()
    pltpu.sync_copy(agg_sp.at[pl.ds(s * _RPT, _RPT)], cnt_out.at[c * _NS + s])


_sc_aggregate = functools.partial(
    pl.kernel,
    out_type=(jax.ShapeDtypeStruct((2 * _NS, _RPT, _DH), jnp.float32),
              jax.ShapeDtypeStruct((2 * _NS, _RPT, _DH), jnp.float32)),
    mesh=plsc.VectorSubcoreMesh(core_axis_name="c", subcore_axis_name="s"),
    scratch_types=[
        pltpu.VMEM((_SEG, _K), jnp.int32),
        pltpu.VMEM((_K,), jnp.int32),
        pltpu.VMEM((_K, _DH), jnp.float32),
        pltpu.VMEM((_K, _DH), jnp.float32),
        pltpu.VMEM_SHARED((_N, _DH), jnp.float32),
        pltpu.SemaphoreType.DMA,
        pltpu.SemaphoreType.DMA,
    ],
)(_sc_agg_body)


_BM = 1000


def _tc_body(a0, a1, c0, c1, xd, wl0, wl1, wr, bias, out):
    r = 1.0 / jnp.maximum(c0[:, 0:1] + c1[:, 0:1], 1.0)
    m = jnp.dot(a0[...] * r, wl0[...], preferred_element_type=jnp.float32)
    m += jnp.dot(a1[...] * r, wl1[...], preferred_element_type=jnp.float32)
    m += jnp.dot(xd[...], wr[...], preferred_element_type=jnp.float32)
    out[...] = m + bias[...]


def _fused_update(aggflat, cntflat, x_dst, W_l, W_r, b):
    nblk = _N // _BM
    return pl.pallas_call(
        _tc_body,
        grid=(nblk,),
        in_specs=[
            pl.BlockSpec((_BM, _DH), lambda i: (i, 0)),
            pl.BlockSpec((_BM, _DH), lambda i: (i + _N // _BM, 0)),
            pl.BlockSpec((_BM, _DH), lambda i: (i, 0)),
            pl.BlockSpec((_BM, _DH), lambda i: (i + _N // _BM, 0)),
            pl.BlockSpec((_BM, _D), lambda i: (i, 0)),
            pl.BlockSpec((_DH, _D), lambda i: (0, 0)),
            pl.BlockSpec((_DH, _D), lambda i: (0, 0)),
            pl.BlockSpec((_D, _D), lambda i: (0, 0)),
            pl.BlockSpec((1, _D), lambda i: (0, 0)),
        ],
        out_specs=pl.BlockSpec((_BM, _D), lambda i: (i, 0)),
        out_shape=jax.ShapeDtypeStruct((_N, _D), jnp.float32),
    )(aggflat, aggflat, cntflat, cntflat, x_dst,
      W_l[:_DH], W_l[_DH:], W_r, b.reshape(1, _D))


def _sage(x_src, x_dst, edge_index, W_l, W_r, b):
    xh = x_src.reshape(2 * _N, _DH)
    srcr = edge_index[0].reshape(_NS, _NSEG, _SEG, _K)
    dstr = edge_index[1].reshape(_NS, _NSEG, _SEG, _K)
    agg3, cnt3 = _sc_aggregate(xh, srcr, dstr)
    aggflat = agg3.reshape(2 * _N, _DH)
    cntflat = cnt3.reshape(2 * _N, _DH)
    return _fused_update(aggflat, cntflat, x_dst, W_l, W_r, b)


def kernel(x_user, x_item, W_l_u2i, W_r_u2i, b_u2i, W_l_i2u, W_r_i2u, b_i2u,
           edge_index_u2i, edge_index_i2u):
    out_item = _sage(x_user, x_item, edge_index_u2i, W_l_u2i, W_r_u2i, b_u2i)
    out_user = _sage(x_item, x_user, edge_index_i2u, W_l_i2u, W_r_i2u, b_i2u)
    return (out_user, out_item)


# speedup vs baseline: 4.5571x; 1.0207x over previous
import functools

import jax
import jax.numpy as jnp
from jax import lax
from jax.experimental import pallas as pl
from jax.experimental.pallas import tpu as pltpu
from jax.experimental.pallas import tpu_sc as plsc

_N = 10000
_D = 256
_DH = _D // 2
_E = 160000
_NS = 16
_K = 80
_SEG = 25
_NSEG = 5
_RPT = _N // _NS


def _sc_agg_body(xh, srcr, dstr, agg_out, cnt_out, idx_v, dst_v, gb0, gb1,
                 agg_sp, sem0, sem1, ssem0, ssem1):
    c = lax.axis_index("c")
    s = lax.axis_index("s")
    zero16 = jnp.zeros((16,), jnp.float32)
    one16 = jnp.ones((16,), jnp.float32)

    def wait_scatter(buf, ssem):
        pltpu.make_async_copy(buf, agg_sp.at[pl.ds(0, _K)], ssem).wait()

    def fill(val):
        def fz(i, _):
            for jj in range(_DH // 16):
                gb0[i, pl.ds(jj * 16, 16)] = val
            return 0
        lax.fori_loop(0, _K, fz, 0)

    def zero_my_rows():
        def fzs(i, _):
            pltpu.sync_copy(gb0, agg_sp.at[pl.ds(s * _RPT + i * _K, _K)])
            return 0
        lax.fori_loop(0, _RPT // _K, fzs, 0)
        rem = _RPT - (_RPT // _K) * _K
        pltpu.sync_copy(gb0.at[pl.ds(0, rem)],
                        agg_sp.at[pl.ds(s * _RPT + (_RPT // _K) * _K, rem)])

    fill(zero16)
    zero_my_rows()
    plsc.subcore_barrier()

    def fseg(g, _):
        pltpu.sync_copy(srcr.at[s, g], idx_v)
        pltpu.sync_copy(dstr.at[s, g], dst_v)

        def ft(i, _):
            for jj in range(_K // 16):
                v = idx_v[i, pl.ds(jj * 16, 16)]
                idx_v[i, pl.ds(jj * 16, 16)] = v * 2 + c
            return 0
        lax.fori_loop(0, _SEG, ft, 0)

        def fb(b, _):
            even = (b % 2) == 0

            @pl.when(b == 0)
            def _():
                pltpu.async_copy(xh.at[idx_v.at[b]], gb0, sem0)

            @pl.when(even)
            def _():
                pltpu.make_async_copy(xh.at[pl.ds(0, _K)], gb0, sem0).wait()
                pltpu.async_copy(gb0, agg_sp.at[dst_v.at[b]], ssem0, add=True)

                @pl.when(b + 1 < _SEG)
                def _():
                    @pl.when(b >= 1)
                    def _():
                        wait_scatter(gb1, ssem1)
                    pltpu.async_copy(xh.at[idx_v.at[b + 1]], gb1, sem1)

            @pl.when(jnp.logical_not(even))
            def _():
                pltpu.make_async_copy(xh.at[pl.ds(0, _K)], gb1, sem1).wait()
                pltpu.async_copy(gb1, agg_sp.at[dst_v.at[b]], ssem1, add=True)

                @pl.when(b + 1 < _SEG)
                def _():
                    wait_scatter(gb0, ssem0)
                    pltpu.async_copy(xh.at[idx_v.at[b + 1]], gb0, sem0)
            return 0
        lax.fori_loop(0, _SEG, fb, 0)
        wait_scatter(gb0, ssem0)
        wait_scatter(gb1, ssem1)
        return 0
    lax.fori_loop(0, _NSEG, fseg, 0)

    plsc.subcore_barrier()
    pltpu.sync_copy(agg_sp.at[pl.ds(s * _RPT, _RPT)], agg_out.at[c * _NS + s])
    plsc.subcore_barrier()

    fill(zero16)
    zero_my_rows()
    fill(one16)
    plsc.subcore_barrier()

    def cseg(g, _):
        pltpu.sync_copy(dstr.at[s, g], dst_v)

        def cb(j, _):
            b = 2 * j + c

            @pl.when(b < _SEG)
            def _():
                pltpu.async_copy(gb0, agg_sp.at[dst_v.at[b]], ssem0, add=True)
            return 0
        lax.fori_loop(0, (_SEG + 1) // 2, cb, 0)
        nw = (_SEG + 1) // 2 - c

        def cdrain(j, _):
            wait_scatter(gb0, ssem0)
            return 0
        lax.fori_loop(0, nw, cdrain, 0)
        return 0
    lax.fori_loop(0, _NSEG, cseg, 0)

    plsc.subcore_barrier()
    pltpu.sync_copy(agg_sp.at[pl.ds(s * _RPT, _RPT)], cnt_out.at[c * _NS + s])


_sc_aggregate = functools.partial(
    pl.kernel,
    out_type=(jax.ShapeDtypeStruct((2 * _NS, _RPT, _DH), jnp.float32),
              jax.ShapeDtypeStruct((2 * _NS, _RPT, _DH), jnp.float32)),
    mesh=plsc.VectorSubcoreMesh(core_axis_name="c", subcore_axis_name="s"),
    scratch_types=[
        pltpu.VMEM((_SEG, _K), jnp.int32),
        pltpu.VMEM((_SEG, _K), jnp.int32),
        pltpu.VMEM((_K, _DH), jnp.float32),
        pltpu.VMEM((_K, _DH), jnp.float32),
        pltpu.VMEM_SHARED((_N, _DH), jnp.float32),
        pltpu.SemaphoreType.DMA,
        pltpu.SemaphoreType.DMA,
        pltpu.SemaphoreType.DMA,
        pltpu.SemaphoreType.DMA,
    ],
)(_sc_agg_body)


_BM = 1000


def _tc_body(a0, a1, c0, c1, xd, wl0, wl1, wr, bias, out):
    r = 1.0 / jnp.maximum(c0[:, 0:1] + c1[:, 0:1], 1.0)
    m = jnp.dot(a0[...] * r, wl0[...], preferred_element_type=jnp.float32)
    m += jnp.dot(a1[...] * r, wl1[...], preferred_element_type=jnp.float32)
    m += jnp.dot(xd[...], wr[...], preferred_element_type=jnp.float32)
    out[...] = m + bias[...]


def _fused_update(aggflat, cntflat, x_dst, W_l, W_r, b):
    nblk = _N // _BM
    return pl.pallas_call(
        _tc_body,
        grid=(nblk,),
        in_specs=[
            pl.BlockSpec((_BM, _DH), lambda i: (i, 0)),
            pl.BlockSpec((_BM, _DH), lambda i: (i + _N // _BM, 0)),
            pl.BlockSpec((_BM, _DH), lambda i: (i, 0)),
            pl.BlockSpec((_BM, _DH), lambda i: (i + _N // _BM, 0)),
            pl.BlockSpec((_BM, _D), lambda i: (i, 0)),
            pl.BlockSpec((_DH, _D), lambda i: (0, 0)),
            pl.BlockSpec((_DH, _D), lambda i: (0, 0)),
            pl.BlockSpec((_D, _D), lambda i: (0, 0)),
            pl.BlockSpec((1, _D), lambda i: (0, 0)),
        ],
        out_specs=pl.BlockSpec((_BM, _D), lambda i: (i, 0)),
        out_shape=jax.ShapeDtypeStruct((_N, _D), jnp.float32),
    )(aggflat, aggflat, cntflat, cntflat, x_dst,
      W_l[:_DH], W_l[_DH:], W_r, b.reshape(1, _D))


def _sage(x_src, x_dst, edge_index, W_l, W_r, b):
    xh = x_src.reshape(2 * _N, _DH)
    srcr = edge_index[0].reshape(_NS, _NSEG, _SEG, _K)
    dstr = edge_index[1].reshape(_NS, _NSEG, _SEG, _K)
    agg3, cnt3 = _sc_aggregate(xh, srcr, dstr)
    aggflat = agg3.reshape(2 * _N, _DH)
    cntflat = cnt3.reshape(2 * _N, _DH)
    return _fused_update(aggflat, cntflat, x_dst, W_l, W_r, b)


def kernel(x_user, x_item, W_l_u2i, W_r_u2i, b_u2i, W_l_i2u, W_r_i2u, b_i2u,
           edge_index_u2i, edge_index_i2u):
    out_item = _sage(x_user, x_item, edge_index_u2i, W_l_u2i, W_r_u2i, b_u2i)
    out_user = _sage(x_item, x_user, edge_index_i2u, W_l_i2u, W_r_i2u, b_i2u)
    return (out_user, out_item)


# speedup vs baseline: 4.5639x; 1.0015x over previous
import functools

import jax
import jax.numpy as jnp
from jax import lax
from jax.experimental import pallas as pl
from jax.experimental.pallas import tpu as pltpu
from jax.experimental.pallas import tpu_sc as plsc

_N = 10000
_D = 256
_DH = _D // 2
_E = 160000
_NS = 16
_K = 80
_SEG = 25
_NSEG = 5
_RPT = _N // _NS


def _sc_agg_body(xh, srcr, dstr, agg_out, cnt_out, idx_v, dst_v, gb0, gb1,
                 agg_sp, sem0, sem1, ssem0, ssem1):
    c = lax.axis_index("c")
    s = lax.axis_index("s")
    zero16 = jnp.zeros((16,), jnp.float32)
    one16 = jnp.ones((16,), jnp.float32)

    def wait_scatter(buf, ssem):
        pltpu.make_async_copy(buf, agg_sp.at[pl.ds(0, _K)], ssem).wait()

    def fill(val):
        def fz(i, _):
            for jj in range(_DH // 16):
                gb0[i, pl.ds(jj * 16, 16)] = val
            return 0
        lax.fori_loop(0, _K, fz, 0)

    def zero_my_rows():
        def fzs(i, _):
            pltpu.sync_copy(gb0, agg_sp.at[pl.ds(s * _RPT + i * _K, _K)])
            return 0
        lax.fori_loop(0, _RPT // _K, fzs, 0)
        rem = _RPT - (_RPT // _K) * _K
        pltpu.sync_copy(gb0.at[pl.ds(0, rem)],
                        agg_sp.at[pl.ds(s * _RPT + (_RPT // _K) * _K, rem)])

    fill(zero16)
    zero_my_rows()
    plsc.subcore_barrier()

    def fseg(g, _):
        pltpu.sync_copy(srcr.at[s, g], idx_v)
        pltpu.sync_copy(dstr.at[s, g], dst_v)

        def ft(i, _):
            for jj in range(_K // 16):
                v = idx_v[i, pl.ds(jj * 16, 16)]
                idx_v[i, pl.ds(jj * 16, 16)] = v * 2 + c
            return 0
        lax.fori_loop(0, _SEG, ft, 0)

        def fb(b, _):
            even = (b % 2) == 0

            @pl.when(b == 0)
            def _():
                pltpu.async_copy(xh.at[idx_v.at[b]], gb0, sem0)

            @pl.when(even)
            def _():
                pltpu.make_async_copy(xh.at[pl.ds(0, _K)], gb0, sem0).wait()
                pltpu.async_copy(gb0, agg_sp.at[dst_v.at[b]], ssem0, add=True)

                @pl.when(b + 1 < _SEG)
                def _():
                    @pl.when(b >= 1)
                    def _():
                        wait_scatter(gb1, ssem1)
                    pltpu.async_copy(xh.at[idx_v.at[b + 1]], gb1, sem1)

            @pl.when(jnp.logical_not(even))
            def _():
                pltpu.make_async_copy(xh.at[pl.ds(0, _K)], gb1, sem1).wait()
                pltpu.async_copy(gb1, agg_sp.at[dst_v.at[b]], ssem1, add=True)

                @pl.when(b + 1 < _SEG)
                def _():
                    wait_scatter(gb0, ssem0)
                    pltpu.async_copy(xh.at[idx_v.at[b + 1]], gb0, sem0)
            return 0
        lax.fori_loop(0, _SEG, fb, 0)
        wait_scatter(gb0, ssem0)
        wait_scatter(gb1, ssem1)
        return 0
    lax.fori_loop(0, _NSEG, fseg, 0)

    plsc.subcore_barrier()
    pltpu.sync_copy(agg_sp.at[pl.ds(s * _RPT, _RPT)], agg_out.at[c * _NS + s])
    plsc.subcore_barrier()

    fill(zero16)
    zero_my_rows()
    fill(one16)
    plsc.subcore_barrier()

    def cseg(g, _):
        pltpu.sync_copy(dstr.at[s, g], dst_v)

        def cb(j, _):
            b = 2 * j + c

            @pl.when(b < _SEG)
            def _():
                pltpu.async_copy(gb0, agg_sp.at[dst_v.at[b]], ssem0, add=True)
            return 0
        lax.fori_loop(0, (_SEG + 1) // 2, cb, 0)
        nw = (_SEG + 1) // 2 - c

        def cdrain(j, _):
            wait_scatter(gb0, ssem0)
            return 0
        lax.fori_loop(0, nw, cdrain, 0)
        return 0
    lax.fori_loop(0, _NSEG, cseg, 0)

    plsc.subcore_barrier()
    pltpu.sync_copy(agg_sp.at[pl.ds(s * _RPT, _RPT)], cnt_out.at[c * _NS + s])


_sc_aggregate = functools.partial(
    pl.kernel,
    out_type=(jax.ShapeDtypeStruct((2 * _NS, _RPT, _DH), jnp.float32),
              jax.ShapeDtypeStruct((2 * _NS, _RPT, _DH), jnp.float32)),
    mesh=plsc.VectorSubcoreMesh(core_axis_name="c", subcore_axis_name="s"),
    scratch_types=[
        pltpu.VMEM((_SEG, _K), jnp.int32),
        pltpu.VMEM((_SEG, _K), jnp.int32),
        pltpu.VMEM((_K, _DH), jnp.float32),
        pltpu.VMEM((_K, _DH), jnp.float32),
        pltpu.VMEM_SHARED((_N, _DH), jnp.float32),
        pltpu.SemaphoreType.DMA,
        pltpu.SemaphoreType.DMA,
        pltpu.SemaphoreType.DMA,
        pltpu.SemaphoreType.DMA,
    ],
)(_sc_agg_body)


_BM = 1000


def _tc_body(a0, a1, c0, c1, xd, wl0, wl1, wr, bias, out):
    r = 1.0 / jnp.maximum(c0[:, 0:1] + c1[:, 0:1], 1.0)
    m = jnp.dot(a0[...] * r, wl0[...], preferred_element_type=jnp.float32)
    m += jnp.dot(a1[...] * r, wl1[...], preferred_element_type=jnp.float32)
    m += jnp.dot(xd[...], wr[...], preferred_element_type=jnp.float32)
    out[...] = m + bias[...]


def _fused_update(aggflat, cntflat, x_dst, W_l, W_r, b):
    nblk = _N // _BM
    return pl.pallas_call(
        _tc_body,
        grid=(nblk,),
        in_specs=[
            pl.BlockSpec((_BM, _DH), lambda i: (i, 0)),
            pl.BlockSpec((_BM, _DH), lambda i: (i + _N // _BM, 0)),
            pl.BlockSpec((_BM, _DH), lambda i: (i, 0)),
            pl.BlockSpec((_BM, _DH), lambda i: (i + _N // _BM, 0)),
            pl.BlockSpec((_BM, _D), lambda i: (i, 0)),
            pl.BlockSpec((_DH, _D), lambda i: (0, 0)),
            pl.BlockSpec((_DH, _D), lambda i: (0, 0)),
            pl.BlockSpec((_D, _D), lambda i: (0, 0)),
            pl.BlockSpec((1, _D), lambda i: (0, 0)),
        ],
        out_specs=pl.BlockSpec((_BM, _D), lambda i: (i, 0)),
        out_shape=jax.ShapeDtypeStruct((_N, _D), jnp.float32),
    )(aggflat, aggflat, cntflat, cntflat, x_dst,
      W_l[:_DH], W_l[_DH:], W_r, b.reshape(1, _D))


def _sage(x_src, x_dst, edge_index, W_l, W_r, b):
    xh = x_src.reshape(2 * _N, _DH)
    srcr = edge_index[0].reshape(_NS, _NSEG, _SEG, _K)
    dstr = edge_index[1].reshape(_NS, _NSEG, _SEG, _K)
    agg3, cnt3 = _sc_aggregate(xh, srcr, dstr)
    aggflat = agg3.reshape(2 * _N, _DH)
    cntflat = cnt3.reshape(2 * _N, _DH)
    return _fused_update(aggflat, cntflat, x_dst, W_l, W_r, b)


def kernel(x_user, x_item, W_l_u2i, W_r_u2i, b_u2i, W_l_i2u, W_r_i2u, b_i2u,
           edge_index_u2i, edge_index_i2u):
    agg_a, cnt_a = _sc_aggregate(
        x_user.reshape(2 * _N, _DH),
        edge_index_u2i[0].reshape(_NS, _NSEG, _SEG, _K),
        edge_index_u2i[1].reshape(_NS, _NSEG, _SEG, _K))
    agg_b, cnt_b = _sc_aggregate(
        x_item.reshape(2 * _N, _DH),
        edge_index_i2u[0].reshape(_NS, _NSEG, _SEG, _K),
        edge_index_i2u[1].reshape(_NS, _NSEG, _SEG, _K))
    out_item = _fused_update(agg_a.reshape(2 * _N, _DH),
                             cnt_a.reshape(2 * _N, _DH),
                             x_item, W_l_u2i, W_r_u2i, b_u2i)
    out_user = _fused_update(agg_b.reshape(2 * _N, _DH),
                             cnt_b.reshape(2 * _N, _DH),
                             x_user, W_l_i2u, W_r_i2u, b_i2u)
    return (out_user, out_item)
